# Initial kernel scaffold; baseline (speedup 1.0000x reference)
import jax
import jax.numpy as jnp
from jax.experimental import pallas as pl


def kernel(inputs, W):
    raise NotImplementedError("write your pallas kernel here")


# speedup vs baseline: 1.0802x; 1.0802x over previous
import jax
import jax.numpy as jnp
from jax.experimental import pallas as pl

_TOPK = 8
_BT = 512


def _router_body(x_ref, w_ref, logits_ref, weights_ref, mask_ref):
    x = x_ref[...]
    w = w_ref[...]
    n_exp = w.shape[0]
    bt = x.shape[0]
    logits = jax.lax.dot_general(
        x, w, (((1,), (1,)), ((), ())), preferred_element_type=jnp.float32
    )
    logits_ref[...] = logits

    iota_e = jax.lax.broadcasted_iota(jnp.int32, (bt, n_exp), 1)
    work = logits
    neg = jnp.float32(-jnp.inf)
    vals, idxs = [], []
    for _ in range(_TOPK):
        m = jnp.max(work, axis=1, keepdims=True)
        eq = work == m
        idx = jnp.min(jnp.where(eq, iota_e, n_exp), axis=1, keepdims=True)
        vals.append(m)
        idxs.append(idx)
        work = jnp.where(iota_e == idx, neg, work)
    v = jnp.concatenate(vals, axis=1)
    sel = jnp.concatenate(idxs, axis=1)
    e = jnp.exp(v - v[:, 0:1])
    weights_ref[...] = e / jnp.sum(e, axis=1, keepdims=True)

    sel_t = sel.T
    iota3 = jax.lax.broadcasted_iota(jnp.int32, (n_exp, _TOPK, bt), 0)
    mask_ref[...] = (sel_t[None, :, :] == iota3).astype(jnp.int32)


def kernel(inputs, W):
    b, s, dim = inputs.shape
    n_exp = W.shape[0]
    t = b * s
    x = inputs.reshape(t, dim)
    bt = min(_BT, t)
    grid = (t // bt,)
    weights, mask, logits = pl.pallas_call(
        lambda x_ref, w_ref, wo_ref, mo_ref, lo_ref: _router_body(
            x_ref, w_ref, lo_ref, wo_ref, mo_ref
        ),
        grid=grid,
        in_specs=[
            pl.BlockSpec((bt, dim), lambda i: (i, 0)),
            pl.BlockSpec((n_exp, dim), lambda i: (0, 0)),
        ],
        out_specs=[
            pl.BlockSpec((bt, _TOPK), lambda i: (i, 0)),
            pl.BlockSpec((n_exp, _TOPK, bt), lambda i: (0, 0, i)),
            pl.BlockSpec((bt, n_exp), lambda i: (i, 0)),
        ],
        out_shape=[
            jax.ShapeDtypeStruct((t, _TOPK), jnp.float32),
            jax.ShapeDtypeStruct((n_exp, _TOPK, t), jnp.int32),
            jax.ShapeDtypeStruct((t, n_exp), jnp.float32),
        ],
    )(x, W)
    return (weights, mask, logits)


# speedup vs baseline: 1.4409x; 1.3339x over previous
import jax
import jax.numpy as jnp
from jax.experimental import pallas as pl

_TOPK = 8
_BT = 512


def _router_body(x_ref, w_ref, weights_ref, mask_ref, logits_ref):
    x = x_ref[...]
    w = w_ref[...]
    n_exp = w.shape[0]
    bt = x.shape[0]
    neg = jnp.float32(-jnp.inf)
    logits = jax.lax.dot_general(
        x, w, (((1,), (1,)), ((), ())), preferred_element_type=jnp.float32
    )
    logits_ref[...] = logits
    lt = logits.T

    work = lt
    ms = []
    acc = jnp.zeros((n_exp, bt), jnp.int32)
    for j in range(_TOPK):
        m = jnp.max(work, axis=0, keepdims=True)
        eq = work == m
        onehot = jnp.where(eq, 1, 0)
        mask_ref[:, j, :] = onehot
        acc = acc + onehot
        ms.append(m)
        if j < _TOPK - 1:
            work = jnp.where(eq, neg, work)
    v = jnp.concatenate(ms, axis=0)
    e = jnp.exp(v - v[0:1])
    weights_ref[...] = (e / jnp.sum(e, axis=0, keepdims=True)).T

    total = jnp.sum(acc)

    @pl.when(total != _TOPK * bt)
    def _exact_tie_fallback():
        iota_e = jax.lax.broadcasted_iota(jnp.int32, (n_exp, bt), 0)
        work = lt
        vals = []
        for j in range(_TOPK):
            m = jnp.max(work, axis=0, keepdims=True)
            eq = work == m
            idx = jnp.min(jnp.where(eq, iota_e, n_exp), axis=0, keepdims=True)
            first = iota_e == idx
            mask_ref[:, j, :] = jnp.where(first, 1, 0)
            work = jnp.where(first, neg, work)
            vals.append(m)
        v = jnp.concatenate(vals, axis=0)
        e = jnp.exp(v - v[0:1])
        weights_ref[...] = (e / jnp.sum(e, axis=0, keepdims=True)).T


def kernel(inputs, W):
    b, s, dim = inputs.shape
    n_exp = W.shape[0]
    t = b * s
    x = inputs.reshape(t, dim)
    bt = min(_BT, t)
    grid = (t // bt,)
    weights, mask, logits = pl.pallas_call(
        _router_body,
        grid=grid,
        in_specs=[
            pl.BlockSpec((bt, dim), lambda i: (i, 0)),
            pl.BlockSpec((n_exp, dim), lambda i: (0, 0)),
        ],
        out_specs=[
            pl.BlockSpec((bt, _TOPK), lambda i: (i, 0)),
            pl.BlockSpec((n_exp, _TOPK, bt), lambda i: (0, 0, i)),
            pl.BlockSpec((bt, n_exp), lambda i: (i, 0)),
        ],
        out_shape=[
            jax.ShapeDtypeStruct((t, _TOPK), jnp.float32),
            jax.ShapeDtypeStruct((n_exp, _TOPK, t), jnp.int32),
            jax.ShapeDtypeStruct((t, n_exp), jnp.float32),
        ],
    )(x, W)
    return (weights, mask, logits)


# speedup vs baseline: 1.5008x; 1.0415x over previous
import jax
import jax.numpy as jnp
from jax.experimental import pallas as pl

_TOPK = 8
_BT = 1024


def _router_body(x_ref, w_ref, weights_ref, mask_ref, logits_ref):
    x = x_ref[...]
    w = w_ref[...]
    n_exp = w.shape[0]
    bt = x.shape[0]
    neg = jnp.float32(-jnp.inf)
    logits = jax.lax.dot_general(
        x, w, (((1,), (1,)), ((), ())), preferred_element_type=jnp.float32
    )
    logits_ref[...] = logits
    lt = logits.T

    work = lt
    ms = []
    acc = jnp.zeros((n_exp, bt), jnp.int32)
    for j in range(_TOPK):
        m = jnp.max(work, axis=0, keepdims=True)
        eq = work == m
        onehot = jnp.where(eq, 1, 0)
        mask_ref[:, j, :] = onehot
        acc = acc + onehot
        ms.append(m)
        if j < _TOPK - 1:
            work = jnp.where(eq, neg, work)
    v = jnp.concatenate(ms, axis=0)
    e = jnp.exp(v - v[0:1])
    weights_ref[...] = (e / jnp.sum(e, axis=0, keepdims=True)).T

    total = jnp.sum(acc)

    @pl.when(total != _TOPK * bt)
    def _exact_tie_fallback():
        iota_e = jax.lax.broadcasted_iota(jnp.int32, (n_exp, bt), 0)
        work = lt
        vals = []
        for j in range(_TOPK):
            m = jnp.max(work, axis=0, keepdims=True)
            eq = work == m
            idx = jnp.min(jnp.where(eq, iota_e, n_exp), axis=0, keepdims=True)
            first = iota_e == idx
            mask_ref[:, j, :] = jnp.where(first, 1, 0)
            work = jnp.where(first, neg, work)
            vals.append(m)
        v = jnp.concatenate(vals, axis=0)
        e = jnp.exp(v - v[0:1])
        weights_ref[...] = (e / jnp.sum(e, axis=0, keepdims=True)).T


def kernel(inputs, W):
    b, s, dim = inputs.shape
    n_exp = W.shape[0]
    t = b * s
    x = inputs.reshape(t, dim)
    bt = min(_BT, t)
    grid = (t // bt,)
    weights, mask, logits = pl.pallas_call(
        _router_body,
        grid=grid,
        in_specs=[
            pl.BlockSpec((bt, dim), lambda i: (i, 0)),
            pl.BlockSpec((n_exp, dim), lambda i: (0, 0)),
        ],
        out_specs=[
            pl.BlockSpec((bt, _TOPK), lambda i: (i, 0)),
            pl.BlockSpec((n_exp, _TOPK, bt), lambda i: (0, 0, i)),
            pl.BlockSpec((bt, n_exp), lambda i: (i, 0)),
        ],
        out_shape=[
            jax.ShapeDtypeStruct((t, _TOPK), jnp.float32),
            jax.ShapeDtypeStruct((n_exp, _TOPK, t), jnp.int32),
            jax.ShapeDtypeStruct((t, n_exp), jnp.float32),
        ],
    )(x, W)
    return (weights, mask, logits)
